# baseline (speedup 1.0000x reference)
import jax
import jax.numpy as jnp
from jax.experimental import pallas as pl
from jax.experimental.pallas import tpu as pltpu

S, B, D, F = 4096, 2, 1024, 4096
BM = 1024
BR = BM // B
FK = 1024


def _fused_ffn_kernel(x_ref, r_ref, g_ref, b_ref, w1_ref, b1_ref, w2_ref,
                      b2_ref, o_ref):
    x = x_ref[...].reshape(BM, D)
    xb = x.astype(jnp.bfloat16)

    m = jax.lax.dot_general(
        xb, r_ref[...],
        dimension_numbers=(((1,), (0,)), ((), ())),
        preferred_element_type=jnp.float32)
    e1 = m[:, 0:1]
    logit = m[:, 128:129]
    e2 = jax.lax.dot_general(
        (xb * xb).astype(jnp.bfloat16), r_ref[:, 0:128],
        dimension_numbers=(((1,), (0,)), ((), ())),
        preferred_element_type=jnp.float32)[:, 0:1]

    inv = jax.lax.rsqrt(jnp.maximum(e2 - e1 * e1, 0.0) + 1e-5)
    h = (x - e1) * inv * g_ref[...] + b_ref[...]
    alpha = jax.nn.sigmoid(logit)

    hb = h.astype(jnp.bfloat16)
    h2 = None
    for k in range(F // FK):
        h1 = jax.lax.dot_general(
            hb, w1_ref[:, k * FK:(k + 1) * FK],
            dimension_numbers=(((1,), (0,)), ((), ())),
            preferred_element_type=jnp.float32)
        a = jnp.maximum(h1 + b1_ref[:, k * FK:(k + 1) * FK], 0.0)
        p = jax.lax.dot_general(
            a.astype(jnp.bfloat16), w2_ref[k * FK:(k + 1) * FK, :],
            dimension_numbers=(((1,), (0,)), ((), ())),
            preferred_element_type=jnp.float32)
        h2 = p if h2 is None else h2 + p

    out = x + alpha * (h2 + b2_ref[...])
    o_ref[...] = out.reshape(BR, B, D)


def _cast_kernel(w1_ref, w2_ref, o1_ref, o2_ref):
    o1_ref[...] = w1_ref[...].astype(jnp.bfloat16).T
    o2_ref[...] = w2_ref[...].astype(jnp.bfloat16).T


def _cast_weights(w1, w2):
    n = 8
    return pl.pallas_call(
        _cast_kernel,
        grid=(n,),
        in_specs=[
            pl.BlockSpec((F // n, D), lambda i: (i, 0)),
            pl.BlockSpec((D // n, F), lambda i: (i, 0)),
        ],
        out_specs=[
            pl.BlockSpec((D, F // n), lambda i: (0, i)),
            pl.BlockSpec((F, D // n), lambda i: (0, i)),
        ],
        out_shape=[
            jax.ShapeDtypeStruct((D, F), jnp.bfloat16),
            jax.ShapeDtypeStruct((F, D), jnp.bfloat16),
        ],
        compiler_params=pltpu.CompilerParams(
            dimension_semantics=("arbitrary",),
        ),
    )(w1, w2)


@jax.jit
def _run(x, r, g, b, w1, b1, w2, b2):
    w1, w2 = _cast_weights(w1, w2)
    const = lambda shape: pl.BlockSpec(shape, lambda i: (0, 0))
    return pl.pallas_call(
        _fused_ffn_kernel,
        grid=(S // BR,),
        in_specs=[
            pl.BlockSpec((BR, B, D), lambda i: (i, 0, 0)),
            const((D, 256)),
            const((1, D)),
            const((1, D)),
            const((D, F)),
            const((1, F)),
            const((F, D)),
            const((1, D)),
        ],
        out_specs=pl.BlockSpec((BR, B, D), lambda i: (i, 0, 0)),
        out_shape=jax.ShapeDtypeStruct((S, B, D), jnp.float32),
        compiler_params=pltpu.CompilerParams(
            dimension_semantics=("arbitrary",),
        ),
    )(x, r, g, b, w1, b1, w2, b2)


def kernel(input_features, expert_centroids, ln_g, ln_b, ff1_w, ff1_b, ff2_w,
           ff2_b):
    r = jnp.zeros((D, 256), jnp.bfloat16)
    r = r.at[:, 0].set(jnp.full((D,), 1.0 / D, jnp.bfloat16))
    r = r.at[:, 128].set(expert_centroids[0].astype(jnp.bfloat16))
    return _run(
        input_features,
        r,
        ln_g.reshape(1, D),
        ln_b.reshape(1, D),
        ff1_w,
        ff1_b.reshape(1, F),
        ff2_w,
        ff2_b.reshape(1, D),
    )
